# baseline (speedup 1.0000x reference)
import jax
import jax.numpy as jnp
import numpy as np
from jax.experimental import pallas as pl
from jax.experimental.pallas import tpu as pltpu

N = 10000
E = 320000
F = 128
H = 4
DH = F // H
R = 32
L = 2
G = 16
NE = 10
RMAX = 5.0
SH = 15
INV_AVG = 1.0 / 32.0

BE = 4000
NB = 2000


def _geom_body(ps_ref, pr_ref, geo_ref):
    ps = ps_ref[...]
    pr = pr_ref[...]
    vec = pr - ps
    l2 = jnp.sum(vec * vec, axis=1, keepdims=True)
    length = jnp.sqrt(l2 + 1e-12)
    inv_l = 1.0 / length
    x = vec[:, 0:1] * inv_l
    y = vec[:, 1:2] * inv_l
    z = vec[:, 2:3] * inv_l
    x2 = x * x
    y2 = y * y
    z2 = z * z
    centers = jax.lax.broadcasted_iota(jnp.int32, (1, R), 1).astype(jnp.float32) * (RMAX / (R - 1))
    width = RMAX / R
    d = (length - centers) / width
    geo_ref[:, 0:R] = jnp.exp(-0.5 * d * d).astype(jnp.bfloat16)
    geo_ref[:, 32:33] = (0.4886025119029199 * y).astype(jnp.bfloat16)
    geo_ref[:, 33:34] = (0.4886025119029199 * z).astype(jnp.bfloat16)
    geo_ref[:, 34:35] = (0.4886025119029199 * x).astype(jnp.bfloat16)
    geo_ref[:, 35:36] = (1.0925484305920792 * x * y).astype(jnp.bfloat16)
    geo_ref[:, 36:37] = (1.0925484305920792 * y * z).astype(jnp.bfloat16)
    geo_ref[:, 37:38] = (0.31539156525252005 * (3.0 * z2 - 1.0)).astype(jnp.bfloat16)
    geo_ref[:, 38:39] = (1.0925484305920792 * x * z).astype(jnp.bfloat16)
    geo_ref[:, 39:40] = (0.5462742152960396 * (x2 - y2)).astype(jnp.bfloat16)
    geo_ref[:, 40:41] = (0.5900435899266435 * y * (3.0 * x2 - y2)).astype(jnp.bfloat16)
    geo_ref[:, 41:42] = (2.890611442640554 * x * y * z).astype(jnp.bfloat16)
    geo_ref[:, 42:43] = (0.4570457994644658 * y * (5.0 * z2 - 1.0)).astype(jnp.bfloat16)
    geo_ref[:, 43:44] = (0.3731763325901154 * z * (5.0 * z2 - 3.0)).astype(jnp.bfloat16)
    geo_ref[:, 44:45] = (0.4570457994644658 * x * (5.0 * z2 - 1.0)).astype(jnp.bfloat16)
    geo_ref[:, 45:46] = (1.445305721320277 * z * (x2 - y2)).astype(jnp.bfloat16)
    geo_ref[:, 46:47] = (0.5900435899266435 * x * (x2 - 3.0 * y2)).astype(jnp.bfloat16)
    cut = jnp.where(length < RMAX, 0.5 * (jnp.cos(jnp.pi * length / RMAX) + 1.0), 0.0)
    geo_ref[:, 47:48] = cut.astype(jnp.bfloat16)


def _geometry(psnd, prcv):
    return pl.pallas_call(
        _geom_body,
        grid=(E // BE,),
        in_specs=[
            pl.BlockSpec((BE, 3), lambda i: (i, 0)),
            pl.BlockSpec((BE, 3), lambda i: (i, 0)),
        ],
        out_specs=pl.BlockSpec((BE, 48), lambda i: (i, 0)),
        out_shape=jax.ShapeDtypeStruct((E, 48), jnp.bfloat16),
    )(psnd, prcv)


def _fatt_body(invr_ref, invs_ref, geo_ref, wqkv_ref, wrbf_ref, out_ref):
    qf = jnp.dot(invr_ref[...], wqkv_ref[:, 0:F], preferred_element_type=jnp.float32)
    kv = jnp.dot(invs_ref[...], wqkv_ref[:, F:], preferred_element_type=jnp.float32)
    kf = kv[:, :F]
    vf = kv[:, F:]
    rbf = geo_ref[:, 0:R].astype(jnp.float32)
    cut = geo_ref[:, 47:48].astype(jnp.float32)
    wf = jnp.dot(rbf, wrbf_ref[...], preferred_element_type=jnp.float32)
    t = qf * kf * wf
    scale = cut * (DH ** -0.5)
    for h in range(H):
        sl = slice(h * DH, (h + 1) * DH)
        ah = jnp.sum(t[:, sl], axis=1, keepdims=True) * scale
        out_ref[:, sl] = ah * vf[:, sl]


def _feature_messages(invr, invs, geo, wqkv, wrbf):
    return pl.pallas_call(
        _fatt_body,
        grid=(E // BE,),
        in_specs=[
            pl.BlockSpec((BE, F), lambda i: (i, 0)),
            pl.BlockSpec((BE, F), lambda i: (i, 0)),
            pl.BlockSpec((BE, 48), lambda i: (i, 0)),
            pl.BlockSpec((F, 3 * F), lambda i: (0, 0)),
            pl.BlockSpec((R, F), lambda i: (0, 0)),
        ],
        out_specs=pl.BlockSpec((BE, F), lambda i: (i, 0)),
        out_shape=jax.ShapeDtypeStruct((E, F), jnp.float32),
    )(invr, invs, geo, wqkv, wrbf)


def _eatt_body(invr_ref, invs_ref, geo_ref, wqk_ref, wrbfe_ref, out_ref):
    qe = jnp.dot(invr_ref[...], wqk_ref[:, 0:F], preferred_element_type=jnp.float32)
    ke = jnp.dot(invs_ref[...], wqk_ref[:, F:], preferred_element_type=jnp.float32)
    rbf = geo_ref[:, 0:R].astype(jnp.float32)
    cut = geo_ref[:, 47:48].astype(jnp.float32)
    alpha = jnp.sum(qe * ke, axis=1, keepdims=True) * (F ** -0.5)
    we = jnp.dot(rbf, wrbfe_ref[...], preferred_element_type=jnp.float32)
    ad = alpha * cut * we[:, 0:3]
    out_ref[:, 0:3] = ad[:, 0:1] * geo_ref[:, 32:35].astype(jnp.float32)
    out_ref[:, 3:8] = ad[:, 1:2] * geo_ref[:, 35:40].astype(jnp.float32)
    out_ref[:, 8:15] = ad[:, 2:3] * geo_ref[:, 40:47].astype(jnp.float32)
    out_ref[:, 15:16] = jnp.zeros_like(cut)


def _ev_messages(invr, invs, geo, wqk, wrbfe_pad):
    return pl.pallas_call(
        _eatt_body,
        grid=(E // BE,),
        in_specs=[
            pl.BlockSpec((BE, F), lambda i: (i, 0)),
            pl.BlockSpec((BE, F), lambda i: (i, 0)),
            pl.BlockSpec((BE, 48), lambda i: (i, 0)),
            pl.BlockSpec((F, 2 * F), lambda i: (0, 0)),
            pl.BlockSpec((R, F), lambda i: (0, 0)),
        ],
        out_specs=pl.BlockSpec((BE, 16), lambda i: (i, 0)),
        out_shape=jax.ShapeDtypeStruct((E, 16), jnp.float32),
    )(invr, invs, geo, wqk, wrbfe_pad)


def _upd_body(inv_ref, agg_ref, inv1_ref, invb_ref):
    inv1 = inv_ref[...] + agg_ref[...] * INV_AVG
    inv1_ref[...] = inv1
    invb_ref[...] = inv1.astype(jnp.bfloat16)


def _upd(inv, agg):
    return pl.pallas_call(
        _upd_body,
        grid=(N // NB,),
        in_specs=[
            pl.BlockSpec((NB, F), lambda i: (i, 0)),
            pl.BlockSpec((NB, F), lambda i: (i, 0)),
        ],
        out_specs=[
            pl.BlockSpec((NB, F), lambda i: (i, 0)),
            pl.BlockSpec((NB, F), lambda i: (i, 0)),
        ],
        out_shape=[
            jax.ShapeDtypeStruct((N, F), jnp.float32),
            jax.ShapeDtypeStruct((N, F), jnp.bfloat16),
        ],
    )(inv, agg)


def _mlp_body(inv_ref, ev_ref, agge_ref, w1a_ref, w1b_ref, b1_ref, w2a_ref,
              b2a_ref, w2b_ref, b2b_ref, invo_ref, invb_ref, evo_ref):
    inv1 = inv_ref[...]
    ev1 = ev_ref[...] + agge_ref[...] * INV_AVG
    evn0 = jnp.sum(ev1[:, 0:3] * ev1[:, 0:3], axis=1, keepdims=True)
    evn1 = jnp.sum(ev1[:, 3:8] * ev1[:, 3:8], axis=1, keepdims=True)
    evn2 = jnp.sum(ev1[:, 8:15] * ev1[:, 8:15], axis=1, keepdims=True)
    nb = inv1.shape[0]
    pre = jnp.dot(inv1, w1a_ref[...], preferred_element_type=jnp.float32)
    pre = pre + evn0 * jnp.broadcast_to(w1b_ref[0:1, :], (nb, F))
    pre = pre + evn1 * jnp.broadcast_to(w1b_ref[1:2, :], (nb, F))
    pre = pre + evn2 * jnp.broadcast_to(w1b_ref[2:3, :], (nb, F))
    pre = pre + jnp.broadcast_to(b1_ref[...], (nb, F))
    y = pre * jax.nn.sigmoid(pre)
    o1 = jnp.dot(y, w2a_ref[...], preferred_element_type=jnp.float32)
    o1 = o1 + jnp.broadcast_to(b2a_ref[...], (nb, F))
    o2 = jnp.dot(y, w2b_ref[...], preferred_element_type=jnp.float32)
    o2 = o2 + jnp.broadcast_to(b2b_ref[...], (nb, F))
    invo = inv1 + o1
    invo_ref[...] = invo
    invb_ref[...] = invo.astype(jnp.bfloat16)
    evo_ref[:, 0:3] = ev1[:, 0:3] * (1.0 + o2[:, 0:1])
    evo_ref[:, 3:8] = ev1[:, 3:8] * (1.0 + o2[:, 1:2])
    evo_ref[:, 8:15] = ev1[:, 8:15] * (1.0 + o2[:, 2:3])
    evo_ref[:, 15:16] = jnp.zeros_like(evn0)


def _node_mlp(inv1, ev, agge, w1a, w1b_pad, b1, w2a, b2a, w2b_pad, b2b_pad):
    return pl.pallas_call(
        _mlp_body,
        grid=(N // NB,),
        in_specs=[
            pl.BlockSpec((NB, F), lambda i: (i, 0)),
            pl.BlockSpec((NB, 16), lambda i: (i, 0)),
            pl.BlockSpec((NB, 16), lambda i: (i, 0)),
            pl.BlockSpec((F, F), lambda i: (0, 0)),
            pl.BlockSpec((8, F), lambda i: (0, 0)),
            pl.BlockSpec((1, F), lambda i: (0, 0)),
            pl.BlockSpec((F, F), lambda i: (0, 0)),
            pl.BlockSpec((1, F), lambda i: (0, 0)),
            pl.BlockSpec((F, F), lambda i: (0, 0)),
            pl.BlockSpec((1, F), lambda i: (0, 0)),
        ],
        out_specs=[
            pl.BlockSpec((NB, F), lambda i: (i, 0)),
            pl.BlockSpec((NB, F), lambda i: (i, 0)),
            pl.BlockSpec((NB, 16), lambda i: (i, 0)),
        ],
        out_shape=[
            jax.ShapeDtypeStruct((N, F), jnp.float32),
            jax.ShapeDtypeStruct((N, F), jnp.bfloat16),
            jax.ShapeDtypeStruct((N, 16), jnp.float32),
        ],
    )(inv1, ev, agge, w1a, w1b_pad, b1, w2a, b2a, w2b_pad, b2b_pad)


def _readout_body(inv_ref, wo1_ref, bo1_ref, wo2_ref, out_ref):
    h = jnp.dot(inv_ref[...], wo1_ref[...], preferred_element_type=jnp.float32)
    h = h + jnp.broadcast_to(bo1_ref[...], h.shape)
    h = h * jax.nn.sigmoid(h)
    out_ref[...] = jnp.dot(h, wo2_ref[...], preferred_element_type=jnp.float32)


def _readout(inv, Wo1, bo1, Wo2):
    wo2 = jnp.broadcast_to(Wo2, (F, F))
    out = pl.pallas_call(
        _readout_body,
        grid=(N // NB,),
        in_specs=[
            pl.BlockSpec((NB, F), lambda i: (i, 0)),
            pl.BlockSpec((F, F), lambda i: (0, 0)),
            pl.BlockSpec((1, F), lambda i: (0, 0)),
            pl.BlockSpec((F, F), lambda i: (0, 0)),
        ],
        out_specs=pl.BlockSpec((NB, F), lambda i: (i, 0)),
        out_shape=jax.ShapeDtypeStruct((N, F), jnp.float32),
    )(inv, Wo1, bo1.reshape(1, F), wo2)
    return out[:, :1]


def kernel(positions, node_attrs, edge_index, batch, Wemb, Wq_f, Wk_f, Wv_f,
           Wq_e, Wk_e, Wrbf_f, Wrbf_e, Wex1, bex1, Wex2, bex2, Wo1, bo1, Wo2, bo2):
    snd = edge_index[0]
    rcv = edge_index[1]

    geo = _geometry(positions[snd], positions[rcv])

    inv = node_attrs @ Wemb
    inv_bf = inv.astype(jnp.bfloat16)
    ev = jnp.zeros((N, 16), dtype=jnp.float32)

    wqkv = jnp.concatenate([Wq_f, Wk_f, Wv_f], axis=2).astype(jnp.bfloat16)
    wqke = jnp.concatenate([Wq_e, Wk_e], axis=2).astype(jnp.bfloat16)
    wrbfe_pad = jnp.pad(Wrbf_e, ((0, 0), (0, 0), (0, F - 3)))
    w1a = Wex1[:, :F, :]
    w1b_pad = jnp.pad(Wex1[:, F:, :], ((0, 0), (0, 5), (0, 0)))
    b1 = bex1.reshape(L, 1, F)
    w2a = Wex2[:, :, :F]
    b2a = bex2[:, :F].reshape(L, 1, F)
    w2b_pad = jnp.pad(Wex2[:, :, F:], ((0, 0), (0, 0), (0, F - 3)))
    b2b_pad = jnp.pad(bex2[:, F:], ((0, 0), (0, F - 3))).reshape(L, 1, F)

    for t in range(L):
        invr = inv_bf[rcv]
        invs = inv_bf[snd]
        msg_f = _feature_messages(invr, invs, geo, wqkv[t], Wrbf_f[t])
        agg_f = jax.ops.segment_sum(msg_f, rcv, num_segments=N)

        inv1, inv1_bf = _upd(inv, agg_f)
        inv1r = inv1_bf[rcv]
        inv1s = inv1_bf[snd]
        msg_e = _ev_messages(inv1r, inv1s, geo, wqke[t], wrbfe_pad[t])
        agg_e = jax.ops.segment_sum(msg_e, rcv, num_segments=N)

        inv, inv_bf, ev = _node_mlp(inv1, ev, agg_e, w1a[t], w1b_pad[t], b1[t],
                                    w2a[t], b2a[t], w2b_pad[t], b2b_pad[t])

    e_node = _readout(inv, Wo1, bo1, Wo2) + bo2
    energy = jax.ops.segment_sum(e_node, batch, num_segments=G)
    return energy


# speedup vs baseline: 1.0134x; 1.0134x over previous
import jax
import jax.numpy as jnp
import numpy as np
from jax.experimental import pallas as pl
from jax.experimental.pallas import tpu as pltpu

N = 10000
E = 320000
F = 128
H = 4
DH = F // H
R = 32
L = 2
G = 16
NE = 10
RMAX = 5.0
SH = 15
INV_AVG = 1.0 / 32.0

BE = 4000
NB = 2000


def _geom_body(ps_ref, pr_ref, geo_ref):
    ps = ps_ref[...]
    pr = pr_ref[...]
    vec = pr - ps
    l2 = jnp.sum(vec * vec, axis=1, keepdims=True)
    length = jnp.sqrt(l2 + 1e-12)
    inv_l = 1.0 / length
    x = vec[:, 0:1] * inv_l
    y = vec[:, 1:2] * inv_l
    z = vec[:, 2:3] * inv_l
    x2 = x * x
    y2 = y * y
    z2 = z * z
    centers = jax.lax.broadcasted_iota(jnp.int32, (1, R), 1).astype(jnp.float32) * (RMAX / (R - 1))
    width = RMAX / R
    d = (length - centers) / width
    geo_ref[:, 0:R] = jnp.exp(-0.5 * d * d)
    geo_ref[:, 32:33] = 0.4886025119029199 * y
    geo_ref[:, 33:34] = 0.4886025119029199 * z
    geo_ref[:, 34:35] = 0.4886025119029199 * x
    geo_ref[:, 35:36] = 1.0925484305920792 * x * y
    geo_ref[:, 36:37] = 1.0925484305920792 * y * z
    geo_ref[:, 37:38] = 0.31539156525252005 * (3.0 * z2 - 1.0)
    geo_ref[:, 38:39] = 1.0925484305920792 * x * z
    geo_ref[:, 39:40] = 0.5462742152960396 * (x2 - y2)
    geo_ref[:, 40:41] = 0.5900435899266435 * y * (3.0 * x2 - y2)
    geo_ref[:, 41:42] = 2.890611442640554 * x * y * z
    geo_ref[:, 42:43] = 0.4570457994644658 * y * (5.0 * z2 - 1.0)
    geo_ref[:, 43:44] = 0.3731763325901154 * z * (5.0 * z2 - 3.0)
    geo_ref[:, 44:45] = 0.4570457994644658 * x * (5.0 * z2 - 1.0)
    geo_ref[:, 45:46] = 1.445305721320277 * z * (x2 - y2)
    geo_ref[:, 46:47] = 0.5900435899266435 * x * (x2 - 3.0 * y2)
    cut = jnp.where(length < RMAX, 0.5 * (jnp.cos(jnp.pi * length / RMAX) + 1.0), 0.0)
    geo_ref[:, 47:48] = cut


def _geometry(psnd, prcv):
    return pl.pallas_call(
        _geom_body,
        grid=(E // BE,),
        in_specs=[
            pl.BlockSpec((BE, 3), lambda i: (i, 0)),
            pl.BlockSpec((BE, 3), lambda i: (i, 0)),
        ],
        out_specs=pl.BlockSpec((BE, 48), lambda i: (i, 0)),
        out_shape=jax.ShapeDtypeStruct((E, 48), jnp.float32),
    )(psnd, prcv)


def _fatt_body(qf_ref, kf_ref, vf_ref, geo_ref, wrbf_ref, out_ref):
    rbf = geo_ref[:, 0:R]
    cut = geo_ref[:, 47:48]
    wf = jnp.dot(rbf, wrbf_ref[...], preferred_element_type=jnp.float32)
    qf = qf_ref[...].astype(jnp.float32)
    kf = kf_ref[...].astype(jnp.float32)
    vf = vf_ref[...].astype(jnp.float32)
    t = qf * kf * wf
    scale = cut * (DH ** -0.5)
    for h in range(H):
        sl = slice(h * DH, (h + 1) * DH)
        ah = jnp.sum(t[:, sl], axis=1, keepdims=True) * scale
        out_ref[:, sl] = ah * vf[:, sl]


def _feature_messages(qf, kf, vf, geo, wrbf):
    return pl.pallas_call(
        _fatt_body,
        grid=(E // BE,),
        in_specs=[
            pl.BlockSpec((BE, F), lambda i: (i, 0)),
            pl.BlockSpec((BE, F), lambda i: (i, 0)),
            pl.BlockSpec((BE, F), lambda i: (i, 0)),
            pl.BlockSpec((BE, 48), lambda i: (i, 0)),
            pl.BlockSpec((R, F), lambda i: (0, 0)),
        ],
        out_specs=pl.BlockSpec((BE, F), lambda i: (i, 0)),
        out_shape=jax.ShapeDtypeStruct((E, F), jnp.float32),
    )(qf, kf, vf, geo, wrbf)


def _eatt_body(qe_ref, ke_ref, geo_ref, wrbfe_ref, out_ref):
    rbf = geo_ref[:, 0:R]
    cut = geo_ref[:, 47:48]
    qe = qe_ref[...].astype(jnp.float32)
    ke = ke_ref[...].astype(jnp.float32)
    alpha = jnp.sum(qe * ke, axis=1, keepdims=True) * (F ** -0.5)
    we = jnp.dot(rbf, wrbfe_ref[...], preferred_element_type=jnp.float32)
    ad = alpha * cut * we[:, 0:3]
    out_ref[:, 0:3] = ad[:, 0:1] * geo_ref[:, 32:35]
    out_ref[:, 3:8] = ad[:, 1:2] * geo_ref[:, 35:40]
    out_ref[:, 8:15] = ad[:, 2:3] * geo_ref[:, 40:47]
    out_ref[:, 15:16] = jnp.zeros_like(cut)


def _ev_messages(qe, ke, geo, wrbfe_pad):
    return pl.pallas_call(
        _eatt_body,
        grid=(E // BE,),
        in_specs=[
            pl.BlockSpec((BE, F), lambda i: (i, 0)),
            pl.BlockSpec((BE, F), lambda i: (i, 0)),
            pl.BlockSpec((BE, 48), lambda i: (i, 0)),
            pl.BlockSpec((R, F), lambda i: (0, 0)),
        ],
        out_specs=pl.BlockSpec((BE, 16), lambda i: (i, 0)),
        out_shape=jax.ShapeDtypeStruct((E, 16), jnp.float32),
    )(qe, ke, geo, wrbfe_pad)


def _mm3_body(x_ref, w_ref, q_ref, k_ref, v_ref):
    r = jnp.dot(x_ref[...], w_ref[...], preferred_element_type=jnp.float32)
    q_ref[...] = r[:, :F].astype(jnp.bfloat16)
    k_ref[...] = r[:, F:2 * F].astype(jnp.bfloat16)
    v_ref[...] = r[:, 2 * F:].astype(jnp.bfloat16)


def _node_mm3(x, w):
    return pl.pallas_call(
        _mm3_body,
        grid=(N // NB,),
        in_specs=[
            pl.BlockSpec((NB, F), lambda i: (i, 0)),
            pl.BlockSpec((F, 3 * F), lambda i: (0, 0)),
        ],
        out_specs=[
            pl.BlockSpec((NB, F), lambda i: (i, 0)),
            pl.BlockSpec((NB, F), lambda i: (i, 0)),
            pl.BlockSpec((NB, F), lambda i: (i, 0)),
        ],
        out_shape=[
            jax.ShapeDtypeStruct((N, F), jnp.bfloat16),
            jax.ShapeDtypeStruct((N, F), jnp.bfloat16),
            jax.ShapeDtypeStruct((N, F), jnp.bfloat16),
        ],
    )(x, w)


def _upd_qeke_body(inv_ref, agg_ref, w_ref, inv1_ref, qe_ref, ke_ref):
    inv1 = inv_ref[...] + agg_ref[...] * INV_AVG
    inv1_ref[...] = inv1
    r = jnp.dot(inv1, w_ref[...], preferred_element_type=jnp.float32)
    qe_ref[...] = r[:, :F].astype(jnp.bfloat16)
    ke_ref[...] = r[:, F:].astype(jnp.bfloat16)


def _upd_qeke(inv, agg, wcat):
    return pl.pallas_call(
        _upd_qeke_body,
        grid=(N // NB,),
        in_specs=[
            pl.BlockSpec((NB, F), lambda i: (i, 0)),
            pl.BlockSpec((NB, F), lambda i: (i, 0)),
            pl.BlockSpec((F, 2 * F), lambda i: (0, 0)),
        ],
        out_specs=[
            pl.BlockSpec((NB, F), lambda i: (i, 0)),
            pl.BlockSpec((NB, F), lambda i: (i, 0)),
            pl.BlockSpec((NB, F), lambda i: (i, 0)),
        ],
        out_shape=[
            jax.ShapeDtypeStruct((N, F), jnp.float32),
            jax.ShapeDtypeStruct((N, F), jnp.bfloat16),
            jax.ShapeDtypeStruct((N, F), jnp.bfloat16),
        ],
    )(inv, agg, wcat)


def _mlp_body(inv_ref, ev_ref, agge_ref, w1a_ref, w1b_ref, b1_ref, w2a_ref,
              b2a_ref, w2b_ref, b2b_ref, invo_ref, evo_ref):
    inv1 = inv_ref[...]
    ev1 = ev_ref[...] + agge_ref[...] * INV_AVG
    evn0 = jnp.sum(ev1[:, 0:3] * ev1[:, 0:3], axis=1, keepdims=True)
    evn1 = jnp.sum(ev1[:, 3:8] * ev1[:, 3:8], axis=1, keepdims=True)
    evn2 = jnp.sum(ev1[:, 8:15] * ev1[:, 8:15], axis=1, keepdims=True)
    nb = inv1.shape[0]
    pre = jnp.dot(inv1, w1a_ref[...], preferred_element_type=jnp.float32)
    pre = pre + evn0 * jnp.broadcast_to(w1b_ref[0:1, :], (nb, F))
    pre = pre + evn1 * jnp.broadcast_to(w1b_ref[1:2, :], (nb, F))
    pre = pre + evn2 * jnp.broadcast_to(w1b_ref[2:3, :], (nb, F))
    pre = pre + jnp.broadcast_to(b1_ref[...], (nb, F))
    y = pre * jax.nn.sigmoid(pre)
    o1 = jnp.dot(y, w2a_ref[...], preferred_element_type=jnp.float32)
    o1 = o1 + jnp.broadcast_to(b2a_ref[...], (nb, F))
    o2 = jnp.dot(y, w2b_ref[...], preferred_element_type=jnp.float32)
    o2 = o2 + jnp.broadcast_to(b2b_ref[...], (nb, F))
    invo_ref[...] = inv1 + o1
    evo_ref[:, 0:3] = ev1[:, 0:3] * (1.0 + o2[:, 0:1])
    evo_ref[:, 3:8] = ev1[:, 3:8] * (1.0 + o2[:, 1:2])
    evo_ref[:, 8:15] = ev1[:, 8:15] * (1.0 + o2[:, 2:3])
    evo_ref[:, 15:16] = jnp.zeros_like(evn0)


def _node_mlp(inv1, ev, agge, w1a, w1b_pad, b1, w2a, b2a, w2b_pad, b2b_pad):
    return pl.pallas_call(
        _mlp_body,
        grid=(N // NB,),
        in_specs=[
            pl.BlockSpec((NB, F), lambda i: (i, 0)),
            pl.BlockSpec((NB, 16), lambda i: (i, 0)),
            pl.BlockSpec((NB, 16), lambda i: (i, 0)),
            pl.BlockSpec((F, F), lambda i: (0, 0)),
            pl.BlockSpec((8, F), lambda i: (0, 0)),
            pl.BlockSpec((1, F), lambda i: (0, 0)),
            pl.BlockSpec((F, F), lambda i: (0, 0)),
            pl.BlockSpec((1, F), lambda i: (0, 0)),
            pl.BlockSpec((F, F), lambda i: (0, 0)),
            pl.BlockSpec((1, F), lambda i: (0, 0)),
        ],
        out_specs=[
            pl.BlockSpec((NB, F), lambda i: (i, 0)),
            pl.BlockSpec((NB, 16), lambda i: (i, 0)),
        ],
        out_shape=[
            jax.ShapeDtypeStruct((N, F), jnp.float32),
            jax.ShapeDtypeStruct((N, 16), jnp.float32),
        ],
    )(inv1, ev, agge, w1a, w1b_pad, b1, w2a, b2a, w2b_pad, b2b_pad)


def _readout_body(inv_ref, wo1_ref, bo1_ref, wo2_ref, out_ref):
    h = jnp.dot(inv_ref[...], wo1_ref[...], preferred_element_type=jnp.float32)
    h = h + jnp.broadcast_to(bo1_ref[...], h.shape)
    h = h * jax.nn.sigmoid(h)
    out_ref[...] = jnp.dot(h, wo2_ref[...], preferred_element_type=jnp.float32)


def _readout(inv, Wo1, bo1, Wo2):
    wo2 = jnp.broadcast_to(Wo2, (F, F))
    out = pl.pallas_call(
        _readout_body,
        grid=(N // NB,),
        in_specs=[
            pl.BlockSpec((NB, F), lambda i: (i, 0)),
            pl.BlockSpec((F, F), lambda i: (0, 0)),
            pl.BlockSpec((1, F), lambda i: (0, 0)),
            pl.BlockSpec((F, F), lambda i: (0, 0)),
        ],
        out_specs=pl.BlockSpec((NB, F), lambda i: (i, 0)),
        out_shape=jax.ShapeDtypeStruct((N, F), jnp.float32),
    )(inv, Wo1, bo1.reshape(1, F), wo2)
    return out[:, :1]


def kernel(positions, node_attrs, edge_index, batch, Wemb, Wq_f, Wk_f, Wv_f,
           Wq_e, Wk_e, Wrbf_f, Wrbf_e, Wex1, bex1, Wex2, bex2, Wo1, bo1, Wo2, bo2):
    snd = edge_index[0]
    rcv = edge_index[1]

    geo = _geometry(positions[snd], positions[rcv])

    inv = node_attrs @ Wemb
    ev = jnp.zeros((N, 16), dtype=jnp.float32)

    wrbfe_pad = jnp.pad(Wrbf_e, ((0, 0), (0, 0), (0, F - 3)))
    w1a = Wex1[:, :F, :]
    w1b_pad = jnp.pad(Wex1[:, F:, :], ((0, 0), (0, 5), (0, 0)))
    b1 = bex1.reshape(L, 1, F)
    w2a = Wex2[:, :, :F]
    b2a = bex2[:, :F].reshape(L, 1, F)
    w2b_pad = jnp.pad(Wex2[:, :, F:], ((0, 0), (0, 0), (0, F - 3)))
    b2b_pad = jnp.pad(bex2[:, F:], ((0, 0), (0, F - 3))).reshape(L, 1, F)

    for t in range(L):
        wcat_f = jnp.concatenate([Wq_f[t], Wk_f[t], Wv_f[t]], axis=1)
        q_n, k_n, v_n = _node_mm3(inv, wcat_f)
        qf = q_n[rcv]
        kf = k_n[snd]
        vf = v_n[snd]
        msg_f = _feature_messages(qf, kf, vf, geo, Wrbf_f[t])
        agg_f = jax.ops.segment_sum(msg_f, rcv, num_segments=N)

        wcat_e = jnp.concatenate([Wq_e[t], Wk_e[t]], axis=1)
        inv1, qe_n, ke_n = _upd_qeke(inv, agg_f, wcat_e)
        qe = qe_n[rcv]
        ke = ke_n[snd]
        msg_e = _ev_messages(qe, ke, geo, wrbfe_pad[t])
        agg_e = jax.ops.segment_sum(msg_e, rcv, num_segments=N)

        inv, ev = _node_mlp(inv1, ev, agg_e, w1a[t], w1b_pad[t], b1[t],
                            w2a[t], b2a[t], w2b_pad[t], b2b_pad[t])

    e_node = _readout(inv, Wo1, bo1, Wo2) + bo2
    energy = jax.ops.segment_sum(e_node, batch, num_segments=G)
    return energy
